# baseline (speedup 1.0000x reference)
import functools

import jax
import jax.numpy as jnp
from jax import lax
from jax.experimental import pallas as pl
from jax.experimental.pallas import tpu as pltpu
from jax.experimental.pallas import tpu_sc as plsc

N = 10000
E = 320000
D = 128
L = 4

N_PAD = 10240
ROWS_PER_TILE = N_PAD // 16
CHUNK = 128
NUM_WORKERS = 32
CHUNKS_PER_WORKER = 82
EDGES_PER_WORKER = CHUNKS_PER_WORKER * CHUNK
E_PAD = NUM_WORKERS * EDGES_PER_WORKER


def _seg_sum_body(h_hbm, srci_hbm, dsti_hbm, zeros_hbm, out_hbm,
                  idx_s0, idx_s1, idx_d0, idx_d1, rows0, rows1, agg_sh,
                  isem0, isem1, gsem0, gsem1):
    c = lax.axis_index("c")
    s = lax.axis_index("s")
    wid = c * 16 + s
    base = wid * EDGES_PER_WORKER
    idx_s = (idx_s0, idx_s1)
    idx_d = (idx_d0, idx_d1)
    rows = (rows0, rows1)
    isems = (isem0, isem1)
    gsems = (gsem0, gsem1)

    def load_idx(j, p):
        pltpu.async_copy(srci_hbm.at[pl.ds(base + j * CHUNK, CHUNK)],
                         idx_s[p], isems[p])
        pltpu.async_copy(dsti_hbm.at[pl.ds(base + j * CHUNK, CHUNK)],
                         idx_d[p], isems[p])

    def wait_idx(p):
        pltpu.make_async_copy(srci_hbm.at[pl.ds(0, CHUNK)], idx_s[p],
                              isems[p]).wait()
        pltpu.make_async_copy(dsti_hbm.at[pl.ds(0, CHUNK)], idx_d[p],
                              isems[p]).wait()

    pltpu.sync_copy(zeros_hbm, rows0)
    for k in range(ROWS_PER_TILE // CHUNK):
        pltpu.sync_copy(rows0,
                        agg_sh.at[pl.ds(s * ROWS_PER_TILE + k * CHUNK, CHUNK)])
    plsc.subcore_barrier()

    load_idx(0, 0)
    load_idx(1, 1)
    wait_idx(0)
    pltpu.async_copy(h_hbm.at[idx_s0], rows0, gsem0)

    def step(k, carry):
        for p in range(2):
            j = 2 * k + p

            @pl.when(j + 1 < CHUNKS_PER_WORKER)
            def _():
                wait_idx(1 - p)
                pltpu.async_copy(h_hbm.at[idx_s[1 - p]], rows[1 - p],
                                 gsems[1 - p])
            pltpu.make_async_copy(h_hbm.at[pl.ds(0, CHUNK)], rows[p],
                                  gsems[p]).wait()
            pltpu.sync_copy(rows[p], agg_sh.at[idx_d[p]], add=True)

            @pl.when(j + 2 < CHUNKS_PER_WORKER)
            def _():
                load_idx(j + 2, p)
        return carry

    lax.fori_loop(0, CHUNKS_PER_WORKER // 2, step, 0)
    plsc.subcore_barrier()

    for k in range(ROWS_PER_TILE // CHUNK):
        off = s * ROWS_PER_TILE + k * CHUNK
        pltpu.sync_copy(agg_sh.at[pl.ds(off, CHUNK)], rows0)
        pltpu.sync_copy(rows0, out_hbm.at[pl.ds(c * N_PAD + off, CHUNK)])


_seg_sum = functools.partial(
    pl.kernel,
    out_type=jax.ShapeDtypeStruct((2 * N_PAD, D), jnp.float32),
    mesh=plsc.VectorSubcoreMesh(core_axis_name="c", subcore_axis_name="s"),
    scratch_types=[
        pltpu.VMEM((CHUNK,), jnp.int32),
        pltpu.VMEM((CHUNK,), jnp.int32),
        pltpu.VMEM((CHUNK,), jnp.int32),
        pltpu.VMEM((CHUNK,), jnp.int32),
        pltpu.VMEM((CHUNK, D), jnp.float32),
        pltpu.VMEM((CHUNK, D), jnp.float32),
        pltpu.VMEM_SHARED((N_PAD, D), jnp.float32),
    ] + [pltpu.SemaphoreType.DMA] * 4,
)(_seg_sum_body)


def _layer_body(h_ref, agg_ref, w1_ref, w2_ref, g1_ref, b1_ref, g2_ref,
                b2_ref, out_ref, pool_ref):
    agg = agg_ref[...]
    rst = h_ref[...] + agg[0, :N] + agg[1, :N]
    t = jnp.dot(rst, w1_ref[...], preferred_element_type=jnp.float32)
    mean = jnp.mean(t, axis=0)
    var = jnp.mean((t - mean) ** 2, axis=0)
    t = (t - mean) * lax.rsqrt(var + 1e-5) * g1_ref[...] + b1_ref[...]
    t = jnp.maximum(t, 0.0)
    h2 = jnp.dot(t, w2_ref[...], preferred_element_type=jnp.float32)
    mean2 = jnp.mean(h2, axis=0)
    var2 = jnp.mean((h2 - mean2) ** 2, axis=0)
    h2 = (h2 - mean2) * lax.rsqrt(var2 + 1e-5) * g2_ref[...] + b2_ref[...]
    h2 = jnp.maximum(h2, 0.0)
    out_ref[...] = h2
    pool_ref[...] = jnp.sum(h2, axis=0, keepdims=True)


def _tc_layer(h, agg, w1t, w2t, g1, b1, g2, b2):
    return pl.pallas_call(
        _layer_body,
        out_shape=(
            jax.ShapeDtypeStruct((N, D), jnp.float32),
            jax.ShapeDtypeStruct((1, D), jnp.float32),
        ),
    )(h, agg, w1t, w2t, g1, b1, g2, b2)


def _head_body(x_ref, pools_ref, wt_ref, b_ref, logp_ref, score_ref):
    score = jnp.sum(x_ref[...], axis=0, keepdims=True) @ wt_ref[0]
    score = score + b_ref[0:1, :]
    for i in range(L):
        score = score + pools_ref[i:i + 1, :] @ wt_ref[i + 1] + b_ref[i + 1:i + 2, :]
    m = jnp.max(score)
    lse = jnp.log(jnp.sum(jnp.exp(score - m))) + m
    logp_ref[...] = score - lse
    score_ref[...] = score


def _tc_head(x, pools, pred_wt, pred_b):
    return pl.pallas_call(
        _head_body,
        out_shape=(
            jax.ShapeDtypeStruct((1, D), jnp.float32),
            jax.ShapeDtypeStruct((1, D), jnp.float32),
        ),
    )(x, pools, pred_wt, pred_b)


def kernel(x, edge_index, mlp_W1, mlp_W2, mlp_bn_gamma, mlp_bn_beta,
           bn_gamma, bn_beta, pred_W, pred_b):
    n_fake = E_PAD - E
    src = jnp.concatenate([edge_index[0].astype(jnp.int32),
                           jnp.zeros((n_fake,), jnp.int32)])
    fake_dst = N + jnp.arange(n_fake, dtype=jnp.int32) % (N_PAD - N)
    dst = jnp.concatenate([edge_index[1].astype(jnp.int32), fake_dst])
    zeros = jnp.zeros((CHUNK, D), jnp.float32)
    w1t = mlp_W1.transpose(0, 2, 1)
    w2t = mlp_W2.transpose(0, 2, 1)
    pred_wt = pred_W.transpose(0, 2, 1)

    h = x
    pools = []
    for i in range(L):
        agg = _seg_sum(h, src, dst, zeros).reshape(2, N_PAD, D)
        h, pool = _tc_layer(h, agg, w1t[i], w2t[i], mlp_bn_gamma[i],
                            mlp_bn_beta[i], bn_gamma[i], bn_beta[i])
        pools.append(pool)
    pools = jnp.concatenate(pools, axis=0)
    logp, score = _tc_head(x, pools, pred_wt, pred_b)
    return (logp, score)


# speedup vs baseline: 6.8582x; 6.8582x over previous
import functools

import jax
import jax.numpy as jnp
from jax import lax
from jax.experimental import pallas as pl
from jax.experimental.pallas import tpu as pltpu
from jax.experimental.pallas import tpu_sc as plsc

N = 10000
E = 320000
D = 128
L = 4

N_PAD = 10240
ROWS_PER_TILE = N_PAD // 16
CHUNK = 128
NUM_WORKERS = 32
CHUNKS_PER_WORKER = 82
EDGES_PER_WORKER = CHUNKS_PER_WORKER * CHUNK
E_PAD = NUM_WORKERS * EDGES_PER_WORKER


def _seg_sum_body(h_hbm, srci_hbm, dsti_hbm, zeros_hbm, out_hbm,
                  idx_s0, idx_s1, idx_d0, idx_d1, rows0, rows1, agg_sh,
                  isem0, isem1, gsem0, gsem1):
    c = lax.axis_index("c")
    s = lax.axis_index("s")
    wid = c * 16 + s
    base = wid * EDGES_PER_WORKER
    idx_s = (idx_s0, idx_s1)
    idx_d = (idx_d0, idx_d1)
    rows = (rows0, rows1)
    isems = (isem0, isem1)
    gsems = (gsem0, gsem1)

    def load_idx(j, p):
        pltpu.async_copy(srci_hbm.at[pl.ds(base + j * CHUNK, CHUNK)],
                         idx_s[p], isems[p])
        pltpu.async_copy(dsti_hbm.at[pl.ds(base + j * CHUNK, CHUNK)],
                         idx_d[p], isems[p])

    def wait_idx(p):
        pltpu.make_async_copy(srci_hbm.at[pl.ds(0, CHUNK)], idx_s[p],
                              isems[p]).wait()
        pltpu.make_async_copy(dsti_hbm.at[pl.ds(0, CHUNK)], idx_d[p],
                              isems[p]).wait()

    pltpu.sync_copy(zeros_hbm, rows0)
    for k in range(ROWS_PER_TILE // CHUNK):
        pltpu.sync_copy(rows0,
                        agg_sh.at[pl.ds(s * ROWS_PER_TILE + k * CHUNK, CHUNK)])
    plsc.subcore_barrier()

    load_idx(0, 0)
    load_idx(1, 1)
    wait_idx(0)
    pltpu.async_copy(h_hbm.at[idx_s0], rows0, gsem0)

    def step(k, carry):
        for p in range(2):
            j = 2 * k + p

            @pl.when(j + 1 < CHUNKS_PER_WORKER)
            def _():
                wait_idx(1 - p)
                pltpu.async_copy(h_hbm.at[idx_s[1 - p]], rows[1 - p],
                                 gsems[1 - p])
            pltpu.make_async_copy(h_hbm.at[pl.ds(0, CHUNK)], rows[p],
                                  gsems[p]).wait()
            pltpu.sync_copy(rows[p], agg_sh.at[idx_d[p]], add=True)

            @pl.when(j + 2 < CHUNKS_PER_WORKER)
            def _():
                load_idx(j + 2, p)
        return carry

    lax.fori_loop(0, CHUNKS_PER_WORKER // 2, step, 0)
    plsc.subcore_barrier()

    for k in range(ROWS_PER_TILE // CHUNK):
        off = s * ROWS_PER_TILE + k * CHUNK
        pltpu.sync_copy(agg_sh.at[pl.ds(off, CHUNK)], rows0)
        pltpu.sync_copy(rows0, out_hbm.at[pl.ds(c * N_PAD + off, CHUNK)])


_seg_sum = functools.partial(
    pl.kernel,
    out_type=jax.ShapeDtypeStruct((2 * N_PAD, D), jnp.float32),
    mesh=plsc.VectorSubcoreMesh(core_axis_name="c", subcore_axis_name="s"),
    scratch_types=[
        pltpu.VMEM((CHUNK,), jnp.int32),
        pltpu.VMEM((CHUNK,), jnp.int32),
        pltpu.VMEM((CHUNK,), jnp.int32),
        pltpu.VMEM((CHUNK,), jnp.int32),
        pltpu.VMEM((CHUNK, D), jnp.float32),
        pltpu.VMEM((CHUNK, D), jnp.float32),
        pltpu.VMEM_SHARED((N_PAD, D), jnp.float32),
    ] + [pltpu.SemaphoreType.DMA] * 4,
)(_seg_sum_body)


def _layer_body(h_ref, agg_ref, w1_ref, w2_ref, g1_ref, b1_ref, g2_ref,
                b2_ref, out_ref, pool_ref):
    agg = agg_ref[...]
    rst = h_ref[...] + agg[0, :N] + agg[1, :N]
    t = jnp.dot(rst, w1_ref[...], preferred_element_type=jnp.float32)
    mean = jnp.mean(t, axis=0)
    var = jnp.mean((t - mean) ** 2, axis=0)
    t = (t - mean) * lax.rsqrt(var + 1e-5) * g1_ref[...] + b1_ref[...]
    t = jnp.maximum(t, 0.0)
    h2 = jnp.dot(t, w2_ref[...], preferred_element_type=jnp.float32)
    mean2 = jnp.mean(h2, axis=0)
    var2 = jnp.mean((h2 - mean2) ** 2, axis=0)
    h2 = (h2 - mean2) * lax.rsqrt(var2 + 1e-5) * g2_ref[...] + b2_ref[...]
    h2 = jnp.maximum(h2, 0.0)
    out_ref[...] = h2
    pool_ref[...] = jnp.sum(h2, axis=0, keepdims=True)


def _tc_layer(h, agg, w1t, w2t, g1, b1, g2, b2):
    return pl.pallas_call(
        _layer_body,
        out_shape=(
            jax.ShapeDtypeStruct((N, D), jnp.float32),
            jax.ShapeDtypeStruct((1, D), jnp.float32),
        ),
    )(h, agg, w1t, w2t, g1, b1, g2, b2)


def _head_body(x_ref, pools_ref, wt_ref, b_ref, logp_ref, score_ref):
    score = jnp.sum(x_ref[...], axis=0, keepdims=True) @ wt_ref[0]
    score = score + b_ref[0:1, :]
    for i in range(L):
        score = score + pools_ref[i:i + 1, :] @ wt_ref[i + 1] + b_ref[i + 1:i + 2, :]
    m = jnp.max(score)
    lse = jnp.log(jnp.sum(jnp.exp(score - m))) + m
    logp_ref[...] = score - lse
    score_ref[...] = score


def _tc_head(x, pools, pred_wt, pred_b):
    return pl.pallas_call(
        _head_body,
        out_shape=(
            jax.ShapeDtypeStruct((1, D), jnp.float32),
            jax.ShapeDtypeStruct((1, D), jnp.float32),
        ),
    )(x, pools, pred_wt, pred_b)


def kernel(x, edge_index, mlp_W1, mlp_W2, mlp_bn_gamma, mlp_bn_beta,
           bn_gamma, bn_beta, pred_W, pred_b):
    n_fake = E_PAD - E
    fake_ar = jnp.arange(n_fake, dtype=jnp.int32)
    src = jnp.concatenate([edge_index[0].astype(jnp.int32),
                           (fake_ar * 37) % N])
    fake_dst = N + fake_ar % (N_PAD - N)
    dst = jnp.concatenate([edge_index[1].astype(jnp.int32), fake_dst])
    zeros = jnp.zeros((CHUNK, D), jnp.float32)
    w1t = mlp_W1.transpose(0, 2, 1)
    w2t = mlp_W2.transpose(0, 2, 1)
    pred_wt = pred_W.transpose(0, 2, 1)

    h = x
    pools = []
    for i in range(L):
        agg = _seg_sum(h, src, dst, zeros).reshape(2, N_PAD, D)
        h, pool = _tc_layer(h, agg, w1t[i], w2t[i], mlp_bn_gamma[i],
                            mlp_bn_beta[i], bn_gamma[i], bn_beta[i])
        pools.append(pool)
    pools = jnp.concatenate(pools, axis=0)
    logp, score = _tc_head(x, pools, pred_wt, pred_b)
    return (logp, score)


# speedup vs baseline: 6.9867x; 1.0187x over previous
import functools

import jax
import jax.numpy as jnp
from jax import lax
from jax.experimental import pallas as pl
from jax.experimental.pallas import tpu as pltpu
from jax.experimental.pallas import tpu_sc as plsc

N = 10000
E = 320000
D = 128
L = 4

N_PAD = 10240
ROWS_PER_TILE = N_PAD // 16
CHUNK = 128
NUM_WORKERS = 32
CHUNKS_PER_WORKER = 80
EDGES_PER_WORKER = CHUNKS_PER_WORKER * CHUNK
E_PAD = NUM_WORKERS * EDGES_PER_WORKER


def _seg_sum_body(h_hbm, srci_hbm, dsti_hbm, zeros_hbm, out_hbm,
                  idx_s0, idx_s1, idx_d0, idx_d1, rows0, rows1, agg_sh,
                  isem0, isem1, gsem0, gsem1):
    c = lax.axis_index("c")
    s = lax.axis_index("s")
    wid = c * 16 + s
    base = wid * EDGES_PER_WORKER
    idx_s = (idx_s0, idx_s1)
    idx_d = (idx_d0, idx_d1)
    rows = (rows0, rows1)
    isems = (isem0, isem1)
    gsems = (gsem0, gsem1)

    def load_idx(j, p):
        pltpu.async_copy(srci_hbm.at[pl.ds(base + j * CHUNK, CHUNK)],
                         idx_s[p], isems[p])
        pltpu.async_copy(dsti_hbm.at[pl.ds(base + j * CHUNK, CHUNK)],
                         idx_d[p], isems[p])

    def wait_idx(p):
        pltpu.make_async_copy(srci_hbm.at[pl.ds(0, CHUNK)], idx_s[p],
                              isems[p]).wait()
        pltpu.make_async_copy(dsti_hbm.at[pl.ds(0, CHUNK)], idx_d[p],
                              isems[p]).wait()

    pltpu.sync_copy(zeros_hbm, rows0)
    for k in range(ROWS_PER_TILE // CHUNK):
        pltpu.sync_copy(rows0,
                        agg_sh.at[pl.ds(s * ROWS_PER_TILE + k * CHUNK, CHUNK)])
    plsc.subcore_barrier()

    load_idx(0, 0)
    load_idx(1, 1)
    wait_idx(0)
    pltpu.async_copy(h_hbm.at[idx_s0], rows0, gsem0)

    def step(k, carry):
        for p in range(2):
            j = 2 * k + p

            @pl.when(j + 1 < CHUNKS_PER_WORKER)
            def _():
                wait_idx(1 - p)
                pltpu.async_copy(h_hbm.at[idx_s[1 - p]], rows[1 - p],
                                 gsems[1 - p])
            pltpu.make_async_copy(h_hbm.at[pl.ds(0, CHUNK)], rows[p],
                                  gsems[p]).wait()
            pltpu.sync_copy(rows[p], agg_sh.at[idx_d[p]], add=True)

            @pl.when(j + 2 < CHUNKS_PER_WORKER)
            def _():
                load_idx(j + 2, p)
        return carry

    lax.fori_loop(0, CHUNKS_PER_WORKER // 2, step, 0)
    plsc.subcore_barrier()

    for k in range(ROWS_PER_TILE // CHUNK):
        off = s * ROWS_PER_TILE + k * CHUNK
        pltpu.sync_copy(agg_sh.at[pl.ds(off, CHUNK)], rows0)
        pltpu.sync_copy(rows0, out_hbm.at[pl.ds(c * N_PAD + off, CHUNK)])


_seg_sum = functools.partial(
    pl.kernel,
    out_type=jax.ShapeDtypeStruct((2 * N_PAD, D), jnp.float32),
    mesh=plsc.VectorSubcoreMesh(core_axis_name="c", subcore_axis_name="s"),
    scratch_types=[
        pltpu.VMEM((CHUNK,), jnp.int32),
        pltpu.VMEM((CHUNK,), jnp.int32),
        pltpu.VMEM((CHUNK,), jnp.int32),
        pltpu.VMEM((CHUNK,), jnp.int32),
        pltpu.VMEM((CHUNK, D), jnp.float32),
        pltpu.VMEM((CHUNK, D), jnp.float32),
        pltpu.VMEM_SHARED((N_PAD, D), jnp.float32),
    ] + [pltpu.SemaphoreType.DMA] * 4,
)(_seg_sum_body)


def _layer_body(h_ref, agg_ref, w1_ref, w2_ref, g1_ref, b1_ref, g2_ref,
                b2_ref, out_ref, pool_ref):
    agg = agg_ref[...]
    rst = h_ref[...] + agg[0, :N] + agg[1, :N]
    t = jnp.dot(rst, w1_ref[...], preferred_element_type=jnp.float32)
    mean = jnp.mean(t, axis=0)
    var = jnp.mean((t - mean) ** 2, axis=0)
    t = (t - mean) * lax.rsqrt(var + 1e-5) * g1_ref[...] + b1_ref[...]
    t = jnp.maximum(t, 0.0)
    h2 = jnp.dot(t, w2_ref[...], preferred_element_type=jnp.float32)
    mean2 = jnp.mean(h2, axis=0)
    var2 = jnp.mean((h2 - mean2) ** 2, axis=0)
    h2 = (h2 - mean2) * lax.rsqrt(var2 + 1e-5) * g2_ref[...] + b2_ref[...]
    h2 = jnp.maximum(h2, 0.0)
    out_ref[...] = h2
    pool_ref[...] = jnp.sum(h2, axis=0, keepdims=True)


def _tc_layer(h, agg, w1t, w2t, g1, b1, g2, b2):
    return pl.pallas_call(
        _layer_body,
        out_shape=(
            jax.ShapeDtypeStruct((N, D), jnp.float32),
            jax.ShapeDtypeStruct((1, D), jnp.float32),
        ),
    )(h, agg, w1t, w2t, g1, b1, g2, b2)


def _head_body(x_ref, pools_ref, wt_ref, b_ref, logp_ref, score_ref):
    score = jnp.sum(x_ref[...], axis=0, keepdims=True) @ wt_ref[0]
    score = score + b_ref[0:1, :]
    for i in range(L):
        score = score + pools_ref[i:i + 1, :] @ wt_ref[i + 1] + b_ref[i + 1:i + 2, :]
    m = jnp.max(score)
    lse = jnp.log(jnp.sum(jnp.exp(score - m))) + m
    logp_ref[...] = score - lse
    score_ref[...] = score


def _tc_head(x, pools, pred_wt, pred_b):
    return pl.pallas_call(
        _head_body,
        out_shape=(
            jax.ShapeDtypeStruct((1, D), jnp.float32),
            jax.ShapeDtypeStruct((1, D), jnp.float32),
        ),
    )(x, pools, pred_wt, pred_b)


def kernel(x, edge_index, mlp_W1, mlp_W2, mlp_bn_gamma, mlp_bn_beta,
           bn_gamma, bn_beta, pred_W, pred_b):
    n_fake = E_PAD - E
    fake_ar = jnp.arange(n_fake, dtype=jnp.int32)
    src = jnp.concatenate([edge_index[0].astype(jnp.int32),
                           (fake_ar * 37) % N])
    fake_dst = N + fake_ar % (N_PAD - N)
    dst = jnp.concatenate([edge_index[1].astype(jnp.int32), fake_dst])
    zeros = jnp.zeros((CHUNK, D), jnp.float32)
    w1t = mlp_W1.transpose(0, 2, 1)
    w2t = mlp_W2.transpose(0, 2, 1)
    pred_wt = pred_W.transpose(0, 2, 1)

    h = x
    pools = []
    for i in range(L):
        agg = _seg_sum(h, src, dst, zeros).reshape(2, N_PAD, D)
        h, pool = _tc_layer(h, agg, w1t[i], w2t[i], mlp_bn_gamma[i],
                            mlp_bn_beta[i], bn_gamma[i], bn_beta[i])
        pools.append(pool)
    pools = jnp.concatenate(pools, axis=0)
    logp, score = _tc_head(x, pools, pred_wt, pred_b)
    return (logp, score)
